# baseline (speedup 1.0000x reference)
import jax
import jax.numpy as jnp
from jax.experimental import pallas as pl
from jax.experimental.pallas import tpu as pltpu


def _gelu_exact(v):
    return 0.5 * v * (1.0 + jax.lax.erf(v * 0.7071067811865476))


def _mole_block(x_ref, gW_ref, gb_ref, bW_ref, bb_ref, A_ref, Bf_ref,
                tri_ref, exp_ref, out_ref, probs_ref, wscr_ref, bfscr_ref,
                afscr_ref):
    @pl.when(pl.program_id(0) == 0)
    def _cast_w():
        wscr_ref[...] = bW_ref[...].astype(jnp.bfloat16)
        bfscr_ref[...] = Bf_ref[...].astype(jnp.bfloat16)
        n_e, _, rk = A_ref.shape
        for ee in range(n_e):
            afscr_ref[:, ee * rk:(ee + 1) * rk] = (
                A_ref[ee].astype(jnp.bfloat16))

    xb = x_ref[...]
    xbf = xb.astype(jnp.bfloat16)

    h_pre = jnp.dot(xbf, afscr_ref[...], preferred_element_type=jnp.float32)
    bhalf = wscr_ref.shape[0] // 2
    base_pre1 = jax.lax.dot_general(
        xbf, wscr_ref[:bhalf, :], (((1,), (1,)), ((), ())),
        preferred_element_type=jnp.float32)
    base_pre2 = jax.lax.dot_general(
        xbf, wscr_ref[bhalf:, :], (((1,), (1,)), ((), ())),
        preferred_element_type=jnp.float32)


    logits = jax.lax.dot_general(xb, gW_ref[...], (((1,), (1,)), ((), ())),
                                 preferred_element_type=jnp.float32)
    logits = logits + gb_ref[...]
    m = jnp.max(logits, axis=-1, keepdims=True)
    ex = jnp.exp(logits - m)
    rinv = 1.0 / jnp.sum(ex, axis=-1, keepdims=True)
    probs_ref[...] = ex * rinv
    onehot = (logits >= m).astype(jnp.float32)
    cnt = jnp.dot(onehot, tri_ref[...], preferred_element_type=jnp.float32)
    scale8 = onehot * (cnt == 1.0).astype(jnp.float32) * rinv
    scale = jnp.dot(scale8, exp_ref[...], preferred_element_type=jnp.float32)

    hs = (_gelu_exact(h_pre) * scale).astype(jnp.bfloat16)
    half = out_ref.shape[1] // 2
    lora1 = jnp.dot(hs, bfscr_ref[:, :half],
                    preferred_element_type=jnp.float32)
    base1 = _gelu_exact(base_pre1 + bb_ref[:, :half])
    out_ref[:, :half] = base1 + lora1
    lora2 = jnp.dot(hs, bfscr_ref[:, half:],
                    preferred_element_type=jnp.float32)
    base2 = _gelu_exact(base_pre2 + bb_ref[:, half:])
    out_ref[:, half:] = base2 + lora2


def kernel(x, gate_W, gate_b, base_W, base_b, lora_A, lora_B):
    b, s, d = x.shape
    e, _, r = lora_A.shape
    nt = b * s
    xf = x.reshape(nt, d)
    Af = jnp.transpose(lora_A, (1, 0, 2)).reshape(d, e * r).astype(
        jnp.bfloat16)
    Bf = lora_B.reshape(e * r, d)
    gb = gate_b.reshape(1, e)
    bb = base_b.reshape(1, d)
    tri = jnp.triu(jnp.ones((e, e), jnp.float32))
    expand = (jnp.arange(e * r, dtype=jnp.int32)[None, :] // r
              == jnp.arange(e, dtype=jnp.int32)[:, None]).astype(jnp.float32)

    TB = 512
    ni = nt // TB

    out, probs = pl.pallas_call(
        _mole_block,
        grid=(ni,),
        in_specs=[
            pl.BlockSpec((TB, d), lambda i: (i, 0)),
            pl.BlockSpec((e, d), lambda i: (0, 0)),
            pl.BlockSpec((1, e), lambda i: (0, 0)),
            pl.BlockSpec((d, d), lambda i: (0, 0)),
            pl.BlockSpec((1, d), lambda i: (0, 0)),
            pl.BlockSpec((e, d, r), lambda i: (0, 0, 0)),
            pl.BlockSpec((e * r, d), lambda i: (0, 0)),
            pl.BlockSpec((e, e), lambda i: (0, 0)),
            pl.BlockSpec((e, e * r), lambda i: (0, 0)),
        ],
        out_specs=[
            pl.BlockSpec((TB, d), lambda i: (i, 0)),
            pl.BlockSpec((TB, e), lambda i: (i, 0)),
        ],
        out_shape=[
            jax.ShapeDtypeStruct((nt, d), jnp.float32),
            jax.ShapeDtypeStruct((nt, e), jnp.float32),
        ],
        scratch_shapes=[pltpu.VMEM((d, d), jnp.bfloat16),
                        pltpu.VMEM((e * r, d), jnp.bfloat16),
                        pltpu.VMEM((d, e * r), jnp.bfloat16)],
        compiler_params=pltpu.CompilerParams(
            dimension_semantics=("parallel",),
        ),
    )(xf, gate_W, gb, base_W, bb, lora_A, Bf, tri, expand)
    return out.reshape(b, s, d), probs


# speedup vs baseline: 1.0003x; 1.0003x over previous
import jax
import jax.numpy as jnp
from jax.experimental import pallas as pl
from jax.experimental.pallas import tpu as pltpu


def _gelu_exact(v):
    return 0.5 * v * (1.0 + jax.lax.erf(v * 0.7071067811865476))


def _mole_block(x_ref, gW_ref, gb_ref, bW_ref, bb_ref, Af_ref, Bf_ref,
                tri_ref, exp_ref, out_ref, probs_ref, wscr_ref, bfscr_ref):
    @pl.when(pl.program_id(0) == 0)
    def _cast_w():
        wscr_ref[...] = bW_ref[...].astype(jnp.bfloat16)
        bfscr_ref[...] = Bf_ref[...].astype(jnp.bfloat16)

    xb = x_ref[...]
    xbf = xb.astype(jnp.bfloat16)

    h_pre = jnp.dot(xbf, Af_ref[...], preferred_element_type=jnp.float32)
    bhalf = wscr_ref.shape[0] // 2
    base_pre1 = jax.lax.dot_general(
        xbf, wscr_ref[:bhalf, :], (((1,), (1,)), ((), ())),
        preferred_element_type=jnp.float32)
    base_pre2 = jax.lax.dot_general(
        xbf, wscr_ref[bhalf:, :], (((1,), (1,)), ((), ())),
        preferred_element_type=jnp.float32)


    logits = jax.lax.dot_general(xb, gW_ref[...], (((1,), (1,)), ((), ())),
                                 preferred_element_type=jnp.float32)
    logits = logits + gb_ref[...]
    m = jnp.max(logits, axis=-1, keepdims=True)
    ex = jnp.exp(logits - m)
    rinv = 1.0 / jnp.sum(ex, axis=-1, keepdims=True)
    probs_ref[...] = ex * rinv
    onehot = (logits >= m).astype(jnp.float32)
    cnt = jnp.dot(onehot, tri_ref[...], preferred_element_type=jnp.float32)
    scale8 = onehot * (cnt == 1.0).astype(jnp.float32) * rinv
    scale = jnp.dot(scale8, exp_ref[...], preferred_element_type=jnp.float32)

    hs = (_gelu_exact(h_pre) * scale).astype(jnp.bfloat16)
    half = out_ref.shape[1] // 2
    lora1 = jnp.dot(hs, bfscr_ref[:, :half],
                    preferred_element_type=jnp.float32)
    base1 = _gelu_exact(base_pre1 + bb_ref[:, :half])
    out_ref[:, :half] = base1 + lora1
    lora2 = jnp.dot(hs, bfscr_ref[:, half:],
                    preferred_element_type=jnp.float32)
    base2 = _gelu_exact(base_pre2 + bb_ref[:, half:])
    out_ref[:, half:] = base2 + lora2


def kernel(x, gate_W, gate_b, base_W, base_b, lora_A, lora_B):
    b, s, d = x.shape
    e, _, r = lora_A.shape
    nt = b * s
    xf = x.reshape(nt, d)
    Af = jnp.transpose(lora_A, (1, 0, 2)).reshape(d, e * r).astype(
        jnp.bfloat16)
    Bf = lora_B.reshape(e * r, d)
    gb = gate_b.reshape(1, e)
    bb = base_b.reshape(1, d)
    tri = jnp.triu(jnp.ones((e, e), jnp.float32))
    expand = (jnp.arange(e * r, dtype=jnp.int32)[None, :] // r
              == jnp.arange(e, dtype=jnp.int32)[:, None]).astype(jnp.float32)

    TB = 256
    ni = nt // TB

    out, probs = pl.pallas_call(
        _mole_block,
        grid=(ni,),
        in_specs=[
            pl.BlockSpec((TB, d), lambda i: (i, 0)),
            pl.BlockSpec((e, d), lambda i: (0, 0)),
            pl.BlockSpec((1, e), lambda i: (0, 0)),
            pl.BlockSpec((d, d), lambda i: (0, 0)),
            pl.BlockSpec((1, d), lambda i: (0, 0)),
            pl.BlockSpec((d, e * r), lambda i: (0, 0)),
            pl.BlockSpec((e * r, d), lambda i: (0, 0)),
            pl.BlockSpec((e, e), lambda i: (0, 0)),
            pl.BlockSpec((e, e * r), lambda i: (0, 0)),
        ],
        out_specs=[
            pl.BlockSpec((TB, d), lambda i: (i, 0)),
            pl.BlockSpec((TB, e), lambda i: (i, 0)),
        ],
        out_shape=[
            jax.ShapeDtypeStruct((nt, d), jnp.float32),
            jax.ShapeDtypeStruct((nt, e), jnp.float32),
        ],
        scratch_shapes=[pltpu.VMEM((d, d), jnp.bfloat16),
                        pltpu.VMEM((e * r, d), jnp.bfloat16)],
        compiler_params=pltpu.CompilerParams(
            dimension_semantics=("parallel",),
        ),
    )(xf, gate_W, gb, base_W, bb, Af, Bf, tri, expand)
    return out.reshape(b, s, d), probs


# speedup vs baseline: 1.1085x; 1.1082x over previous
import jax
import jax.numpy as jnp
from jax.experimental import pallas as pl
from jax.experimental.pallas import tpu as pltpu


def _gelu_exact(v):
    return 0.5 * v * (1.0 + jax.lax.erf(v * 0.7071067811865476))


def _mole_block(x_ref, gW_ref, gb_ref, bW_ref, bb_ref, Af_ref, Bf_ref,
                tri_ref, exp_ref, out_ref, probs_ref, wscr_ref, bfscr_ref):
    @pl.when(pl.program_id(0) == 0)
    def _cast_w():
        wscr_ref[...] = bW_ref[...].astype(jnp.bfloat16)
        bfscr_ref[...] = Bf_ref[...].astype(jnp.bfloat16)

    xb = x_ref[...]
    xbf = xb.astype(jnp.bfloat16)

    h_pre = jnp.dot(xbf, Af_ref[...], preferred_element_type=jnp.float32)
    bhalf = wscr_ref.shape[0] // 2
    base_pre1 = jax.lax.dot_general(
        xbf, wscr_ref[:bhalf, :], (((1,), (1,)), ((), ())),
        preferred_element_type=jnp.float32)
    base_pre2 = jax.lax.dot_general(
        xbf, wscr_ref[bhalf:, :], (((1,), (1,)), ((), ())),
        preferred_element_type=jnp.float32)


    logits = jax.lax.dot_general(xb, gW_ref[...], (((1,), (1,)), ((), ())),
                                 preferred_element_type=jnp.float32)
    logits = logits + gb_ref[...]
    m = jnp.max(logits, axis=-1, keepdims=True)
    ex = jnp.exp(logits - m)
    rinv = 1.0 / jnp.sum(ex, axis=-1, keepdims=True)
    probs_ref[...] = ex * rinv
    onehot = (logits >= m).astype(jnp.float32)
    cnt = jnp.dot(onehot, tri_ref[...], preferred_element_type=jnp.float32)
    scale8 = onehot * (cnt == 1.0).astype(jnp.float32) * rinv
    scale = jnp.dot(scale8, exp_ref[...], preferred_element_type=jnp.float32)

    hs = (_gelu_exact(h_pre) * scale).astype(jnp.bfloat16)
    half = out_ref.shape[1] // 2
    lora1 = jnp.dot(hs, bfscr_ref[:, :half],
                    preferred_element_type=jnp.float32)
    base1 = _gelu_exact(base_pre1 + bb_ref[:, :half])
    out_ref[:, :half] = base1 + lora1
    lora2 = jnp.dot(hs, bfscr_ref[:, half:],
                    preferred_element_type=jnp.float32)
    base2 = _gelu_exact(base_pre2 + bb_ref[:, half:])
    out_ref[:, half:] = base2 + lora2


def kernel(x, gate_W, gate_b, base_W, base_b, lora_A, lora_B):
    b, s, d = x.shape
    e, _, r = lora_A.shape
    nt = b * s
    xf = x.reshape(nt, d)
    Af = jnp.transpose(lora_A, (1, 0, 2)).reshape(d, e * r).astype(
        jnp.bfloat16)
    Bf = lora_B.reshape(e * r, d)
    gb = gate_b.reshape(1, e)
    bb = base_b.reshape(1, d)
    tri = jnp.triu(jnp.ones((e, e), jnp.float32))
    expand = (jnp.arange(e * r, dtype=jnp.int32)[None, :] // r
              == jnp.arange(e, dtype=jnp.int32)[:, None]).astype(jnp.float32)

    TB = 512
    ni = nt // TB

    out, probs = pl.pallas_call(
        _mole_block,
        grid=(ni,),
        in_specs=[
            pl.BlockSpec((TB, d), lambda i: (i, 0)),
            pl.BlockSpec((e, d), lambda i: (0, 0)),
            pl.BlockSpec((1, e), lambda i: (0, 0)),
            pl.BlockSpec((d, d), lambda i: (0, 0)),
            pl.BlockSpec((1, d), lambda i: (0, 0)),
            pl.BlockSpec((d, e * r), lambda i: (0, 0)),
            pl.BlockSpec((e * r, d), lambda i: (0, 0)),
            pl.BlockSpec((e, e), lambda i: (0, 0)),
            pl.BlockSpec((e, e * r), lambda i: (0, 0)),
        ],
        out_specs=[
            pl.BlockSpec((TB, d), lambda i: (i, 0)),
            pl.BlockSpec((TB, e), lambda i: (i, 0)),
        ],
        out_shape=[
            jax.ShapeDtypeStruct((nt, d), jnp.float32),
            jax.ShapeDtypeStruct((nt, e), jnp.float32),
        ],
        scratch_shapes=[pltpu.VMEM((d, d), jnp.bfloat16),
                        pltpu.VMEM((e * r, d), jnp.bfloat16)],
        compiler_params=pltpu.CompilerParams(
            dimension_semantics=("parallel",),
        ),
    )(xf, gate_W, gb, base_W, bb, Af, Bf, tri, expand)
    return out.reshape(b, s, d), probs
